# baseline (speedup 1.0000x reference)
import functools

import jax
import jax.numpy as jnp
from jax.experimental import pallas as pl
from jax.experimental.pallas import tpu as pltpu


def _manual_kernel(x_ref, o_ref, scratch, sem, *, nsteps, nbuf):
    c = pl.program_id(0)
    j = pl.program_id(1)
    b = c * nsteps + j
    buf = jax.lax.rem(j, nbuf)

    @pl.when(j >= nbuf)
    def _():
        pltpu.make_async_copy(scratch.at[buf], o_ref.at[b - nbuf],
                              sem.at[buf]).wait()

    x = x_ref[0]
    L = x.shape[0]
    row = jax.lax.broadcasted_iota(jnp.int32, (L, L), 0)
    col = jax.lax.broadcasted_iota(jnp.int32, (L, L), 1)
    tri_incl = (col <= row).astype(x.dtype)
    csum = jnp.dot(tri_incl, x, preferred_element_type=jnp.float32)
    cshift = csum - x.astype(jnp.float32)
    inv_denom = 1.0 / (jnp.abs(col - row) + 1).astype(jnp.float32)
    scratch[buf] = ((csum[None, :, :] - cshift[:, None, :])
                    * inv_denom[:, :, None]).astype(o_ref.dtype)

    pltpu.make_async_copy(scratch.at[buf], o_ref.at[b], sem.at[buf]).start()

    @pl.when(j == nsteps - 1)
    def _():
        for k in range(nbuf):
            jj = j - (nbuf - 1) + k
            pltpu.make_async_copy(scratch.at[jax.lax.rem(jj, nbuf)],
                                  o_ref.at[c * nsteps + jj],
                                  sem.at[jax.lax.rem(jj, nbuf)]).wait()


def kernel(seq_hiddens):
    B, L, H = seq_hiddens.shape
    out_dtype = seq_hiddens.dtype
    out_itemsize = jnp.dtype(out_dtype).itemsize

    ncores = 2 if B % 2 == 0 else 1
    nsteps = B // ncores
    nbuf = min(4, nsteps)

    out_bytes = B * L * L * H * out_itemsize
    cost = pl.CostEstimate(flops=3 * B * L * L * H + 2 * B * L * L * H,
                           transcendentals=0,
                           bytes_accessed=out_bytes + B * L * H * out_itemsize)

    kern = functools.partial(_manual_kernel, nsteps=nsteps, nbuf=nbuf)
    return pl.pallas_call(
        kern,
        out_shape=jax.ShapeDtypeStruct((B, L, L, H), out_dtype),
        grid=(ncores, nsteps),
        in_specs=[pl.BlockSpec((1, L, H), lambda c, j: (c * nsteps + j, 0, 0))],
        out_specs=pl.BlockSpec(memory_space=pl.ANY),
        scratch_shapes=[
            pltpu.VMEM((nbuf, L, L, H), out_dtype),
            pltpu.SemaphoreType.DMA((nbuf,)),
        ],
        compiler_params=pltpu.CompilerParams(
            dimension_semantics=("parallel", "arbitrary"),
            vmem_limit_bytes=48 << 20),
        cost_estimate=cost,
    )(seq_hiddens)


# speedup vs baseline: 1.0094x; 1.0094x over previous
import jax
import jax.numpy as jnp
from jax.experimental import pallas as pl
from jax.experimental.pallas import tpu as pltpu


def _fused_phrase_kernel(x_ref, o_ref):
    nb, L, _ = x_ref.shape
    row = jax.lax.broadcasted_iota(jnp.int32, (L, L), 0)
    col = jax.lax.broadcasted_iota(jnp.int32, (L, L), 1)
    inv_denom = 1.0 / (jnp.abs(col - row) + 1).astype(jnp.float32)
    for b in range(nb):
        x = x_ref[b]
        tri_incl = (col <= row).astype(x.dtype)
        csum = jnp.dot(tri_incl, x, preferred_element_type=jnp.float32)
        cshift = csum - x.astype(jnp.float32)
        o_ref[b] = ((csum[None, :, :] - cshift[:, None, :])
                    * inv_denom[:, :, None]).astype(o_ref.dtype)


def kernel(seq_hiddens):
    B, L, H = seq_hiddens.shape
    out_dtype = seq_hiddens.dtype
    out_itemsize = jnp.dtype(out_dtype).itemsize

    out_bytes = B * L * L * H * out_itemsize
    cost = pl.CostEstimate(flops=3 * B * L * L * H + 2 * B * L * L * H,
                           transcendentals=0,
                           bytes_accessed=out_bytes + B * L * H * out_itemsize)

    nb = 2 if B % 2 == 0 else 1
    return pl.pallas_call(
        _fused_phrase_kernel,
        out_shape=jax.ShapeDtypeStruct((B, L, L, H), out_dtype),
        grid=(B // nb,),
        in_specs=[pl.BlockSpec((nb, L, H), lambda b: (b, 0, 0))],
        out_specs=pl.BlockSpec((nb, L, L, H), lambda b: (b, 0, 0, 0)),
        compiler_params=pltpu.CompilerParams(
            dimension_semantics=("parallel",),
            vmem_limit_bytes=60 << 20),
        cost_estimate=cost,
    )(seq_hiddens)
